# baseline (speedup 1.0000x reference)
import functools

import jax
import jax.numpy as jnp
from jax import lax
from jax.experimental import pallas as pl
from jax.experimental.pallas import tpu as pltpu
from jax.experimental.pallas import tpu_sc as plsc

N = 10000
E = 320000
D = 128

NC = 2
NS = 16
NW = NC * NS

CHUNK = 64
NITER = 157
E_PER_W = CHUNK * NITER
E_PAD = E_PER_W * NW
N_PAD = 10112
STRIPE = N_PAD // NS
DST_PAD = N


_sc_mesh = plsc.VectorSubcoreMesh(core_axis_name="c", subcore_axis_name="s")


@functools.partial(
    pl.kernel,
    out_type=jax.ShapeDtypeStruct((NC, N_PAD, D), jnp.float32),
    mesh=_sc_mesh,
    scratch_types=[
        pltpu.VMEM((NITER, CHUNK), jnp.int32),
        pltpu.VMEM((NITER, CHUNK), jnp.int32),
        pltpu.VMEM((CHUNK, D), jnp.float32),
        pltpu.VMEM_SHARED((N_PAD, D), jnp.float32),
        pltpu.SemaphoreType.DMA,
    ],
)
def _sc_aggregate(x_hbm, src_hbm, dst_hbm, zeros_hbm, out_hbm,
                  src_v, dst_v, rows_v, agg_sh, sem):
    cid = lax.axis_index("c")
    sid = lax.axis_index("s")
    wid = sid * NC + cid

    pltpu.sync_copy(zeros_hbm, agg_sh.at[pl.ds(sid * STRIPE, STRIPE)])
    pltpu.sync_copy(src_hbm.at[wid], src_v)
    pltpu.sync_copy(dst_hbm.at[wid], dst_v)
    plsc.subcore_barrier()

    def body(i, _):
        pltpu.async_copy(x_hbm.at[src_v.at[i]], rows_v, sem).wait()
        pltpu.sync_copy(rows_v, agg_sh.at[dst_v.at[i]], add=True)
        return ()

    lax.fori_loop(0, NITER, body, (), unroll=False)

    plsc.subcore_barrier()
    pltpu.sync_copy(
        agg_sh.at[pl.ds(sid * STRIPE, STRIPE)],
        out_hbm.at[cid, pl.ds(sid * STRIPE, STRIPE)],
    )


_BLK = 632


def _tc_linear_body(agg_ref, w_ref, b_ref, o_ref):
    a = agg_ref[0] + agg_ref[1]
    h = lax.dot_general(a, w_ref[...], (((1,), (1,)), ((), ())),
                        preferred_element_type=jnp.float32)
    o_ref[...] = jnp.maximum(h + b_ref[...], 0.0)


def _tc_linear(agg2, W, b):
    return pl.pallas_call(
        _tc_linear_body,
        grid=(N_PAD // _BLK,),
        in_specs=[
            pl.BlockSpec((NC, _BLK, D), lambda i: (0, i, 0)),
            pl.BlockSpec((D, D), lambda i: (0, 0)),
            pl.BlockSpec((1, D), lambda i: (0, 0)),
        ],
        out_specs=pl.BlockSpec((_BLK, D), lambda i: (i, 0)),
        out_shape=jax.ShapeDtypeStruct((N_PAD, D), jnp.float32),
    )(agg2, W, b.reshape(1, D))


def kernel(x, edge_index, W, b):
    ei = edge_index.astype(jnp.int32)
    pad = E_PAD - E
    pad_dst = DST_PAD + jnp.arange(pad, dtype=jnp.int32) % (N_PAD - N)
    src = jnp.concatenate([ei[0], jnp.zeros((pad,), jnp.int32)])
    dst = jnp.concatenate([ei[1], pad_dst])
    src = src.reshape(NW, NITER, CHUNK)
    dst = dst.reshape(NW, NITER, CHUNK)
    zeros = jnp.zeros((STRIPE, D), jnp.float32)
    agg2 = _sc_aggregate(x, src, dst, zeros)
    return _tc_linear(agg2, W, b)[:N]


# speedup vs baseline: 1.4392x; 1.4392x over previous
import functools

import jax
import jax.numpy as jnp
from jax import lax
from jax.experimental import pallas as pl
from jax.experimental.pallas import tpu as pltpu
from jax.experimental.pallas import tpu_sc as plsc

N = 10000
E = 320000
D = 128

NC = 2
NS = 16
NW = NC * NS

CHUNK = 100
NITER = 100
E_PER_W = CHUNK * NITER
E_PAD = E_PER_W * NW
N_PAD = 10112
STRIPE = N_PAD // NS
DST_PAD = N


_sc_mesh = plsc.VectorSubcoreMesh(core_axis_name="c", subcore_axis_name="s")


@functools.partial(
    pl.kernel,
    out_type=jax.ShapeDtypeStruct((NC, N_PAD, D), jnp.float32),
    mesh=_sc_mesh,
    scratch_types=[
        pltpu.VMEM((NITER, CHUNK), jnp.int32),
        pltpu.VMEM((NITER, CHUNK), jnp.int32),
        pltpu.VMEM((CHUNK, D), jnp.float32),
        pltpu.VMEM_SHARED((N_PAD, D), jnp.float32),
        pltpu.SemaphoreType.DMA,
    ],
)
def _sc_aggregate(x_hbm, src_hbm, dst_hbm, zeros_hbm, out_hbm,
                  src_v, dst_v, rows_v, agg_sh, sem):
    cid = lax.axis_index("c")
    sid = lax.axis_index("s")
    wid = sid * NC + cid

    pltpu.sync_copy(zeros_hbm, agg_sh.at[pl.ds(sid * STRIPE, STRIPE)])
    pltpu.sync_copy(src_hbm.at[wid], src_v)
    pltpu.sync_copy(dst_hbm.at[wid], dst_v)
    plsc.subcore_barrier()

    def body(i, _):
        pltpu.async_copy(x_hbm.at[src_v.at[i]], rows_v, sem).wait()
        pltpu.sync_copy(rows_v, agg_sh.at[dst_v.at[i]], add=True)
        return ()

    lax.fori_loop(0, NITER, body, (), unroll=False)

    plsc.subcore_barrier()
    pltpu.sync_copy(
        agg_sh.at[pl.ds(sid * STRIPE, STRIPE)],
        out_hbm.at[cid, pl.ds(sid * STRIPE, STRIPE)],
    )


_BLK = 632


def _tc_linear_body(agg_ref, w_ref, b_ref, o_ref):
    a = agg_ref[0] + agg_ref[1]
    h = lax.dot_general(a, w_ref[...], (((1,), (1,)), ((), ())),
                        preferred_element_type=jnp.float32)
    o_ref[...] = jnp.maximum(h + b_ref[...], 0.0)


def _tc_linear(agg2, W, b):
    return pl.pallas_call(
        _tc_linear_body,
        grid=(N_PAD // _BLK,),
        in_specs=[
            pl.BlockSpec((NC, _BLK, D), lambda i: (0, i, 0)),
            pl.BlockSpec((D, D), lambda i: (0, 0)),
            pl.BlockSpec((1, D), lambda i: (0, 0)),
        ],
        out_specs=pl.BlockSpec((_BLK, D), lambda i: (i, 0)),
        out_shape=jax.ShapeDtypeStruct((N_PAD, D), jnp.float32),
    )(agg2, W, b.reshape(1, D))


def kernel(x, edge_index, W, b):
    ei = edge_index.astype(jnp.int32)
    pad = E_PAD - E
    pad_dst = DST_PAD + jnp.arange(pad, dtype=jnp.int32) % (N_PAD - N)
    src = jnp.concatenate([ei[0], jnp.zeros((pad,), jnp.int32)])
    dst = jnp.concatenate([ei[1], pad_dst])
    src = src.reshape(NW, NITER, CHUNK)
    dst = dst.reshape(NW, NITER, CHUNK)
    zeros = jnp.zeros((STRIPE, D), jnp.float32)
    agg2 = _sc_aggregate(x, src, dst, zeros)
    return _tc_linear(agg2, W, b)[:N]


# speedup vs baseline: 1.5069x; 1.0470x over previous
import functools

import jax
import jax.numpy as jnp
from jax import lax
from jax.experimental import pallas as pl
from jax.experimental.pallas import tpu as pltpu
from jax.experimental.pallas import tpu_sc as plsc

N = 10000
E = 320000
D = 128

NC = 2
NS = 16
NW = NC * NS

CHUNK = 125
NITER = 80
E_PER_W = CHUNK * NITER
E_PAD = E_PER_W * NW
N_PAD = 10112
STRIPE = N_PAD // NS
DST_PAD = N


_sc_mesh = plsc.VectorSubcoreMesh(core_axis_name="c", subcore_axis_name="s")


@functools.partial(
    pl.kernel,
    out_type=jax.ShapeDtypeStruct((NC, N_PAD, D), jnp.float32),
    mesh=_sc_mesh,
    scratch_types=[
        pltpu.VMEM((NITER, CHUNK), jnp.int32),
        pltpu.VMEM((NITER, CHUNK), jnp.int32),
        pltpu.VMEM((CHUNK, D), jnp.float32),
        pltpu.VMEM_SHARED((N_PAD, D), jnp.float32),
        pltpu.SemaphoreType.DMA,
    ],
)
def _sc_aggregate(x_hbm, src_hbm, dst_hbm, zeros_hbm, out_hbm,
                  src_v, dst_v, rows_v, agg_sh, sem):
    cid = lax.axis_index("c")
    sid = lax.axis_index("s")
    wid = sid * NC + cid

    pltpu.sync_copy(zeros_hbm, agg_sh.at[pl.ds(sid * STRIPE, STRIPE)])
    pltpu.sync_copy(src_hbm.at[wid], src_v)
    pltpu.sync_copy(dst_hbm.at[wid], dst_v)
    plsc.subcore_barrier()

    def body(i, _):
        pltpu.async_copy(x_hbm.at[src_v.at[i]], rows_v, sem).wait()
        pltpu.sync_copy(rows_v, agg_sh.at[dst_v.at[i]], add=True)
        return ()

    lax.fori_loop(0, NITER, body, (), unroll=False)

    plsc.subcore_barrier()
    pltpu.sync_copy(
        agg_sh.at[pl.ds(sid * STRIPE, STRIPE)],
        out_hbm.at[cid, pl.ds(sid * STRIPE, STRIPE)],
    )


_BLK = 632


def _tc_linear_body(agg_ref, w_ref, b_ref, o_ref):
    a = agg_ref[0] + agg_ref[1]
    h = lax.dot_general(a, w_ref[...], (((1,), (1,)), ((), ())),
                        preferred_element_type=jnp.float32)
    o_ref[...] = jnp.maximum(h + b_ref[...], 0.0)


def _tc_linear(agg2, W, b):
    return pl.pallas_call(
        _tc_linear_body,
        grid=(N_PAD // _BLK,),
        in_specs=[
            pl.BlockSpec((NC, _BLK, D), lambda i: (0, i, 0)),
            pl.BlockSpec((D, D), lambda i: (0, 0)),
            pl.BlockSpec((1, D), lambda i: (0, 0)),
        ],
        out_specs=pl.BlockSpec((_BLK, D), lambda i: (i, 0)),
        out_shape=jax.ShapeDtypeStruct((N_PAD, D), jnp.float32),
    )(agg2, W, b.reshape(1, D))


def kernel(x, edge_index, W, b):
    ei = edge_index.astype(jnp.int32)
    pad = E_PAD - E
    pad_dst = DST_PAD + jnp.arange(pad, dtype=jnp.int32) % (N_PAD - N)
    src = jnp.concatenate([ei[0], jnp.zeros((pad,), jnp.int32)])
    dst = jnp.concatenate([ei[1], pad_dst])
    src = src.reshape(NW, NITER, CHUNK)
    dst = dst.reshape(NW, NITER, CHUNK)
    zeros = jnp.zeros((STRIPE, D), jnp.float32)
    agg2 = _sc_aggregate(x, src, dst, zeros)
    return _tc_linear(agg2, W, b)[:N]


# speedup vs baseline: 1.8672x; 1.2391x over previous
import functools

import jax
import jax.numpy as jnp
from jax import lax
from jax.experimental import pallas as pl
from jax.experimental.pallas import tpu as pltpu
from jax.experimental.pallas import tpu_sc as plsc

N = 10000
E = 320000
D = 128

NC = 2
NS = 16
NW = NC * NS

CHUNK = 125
NITER = 80
NHALF = NITER // 2
E_PER_W = CHUNK * NITER
E_PAD = E_PER_W * NW
N_PAD = 10112
STRIPE = N_PAD // NS
DST_PAD = N


_sc_mesh = plsc.VectorSubcoreMesh(core_axis_name="c", subcore_axis_name="s")


@functools.partial(
    pl.kernel,
    out_type=jax.ShapeDtypeStruct((NC, N_PAD, D), jnp.float32),
    mesh=_sc_mesh,
    scratch_types=[
        pltpu.VMEM((NHALF, CHUNK), jnp.int32),
        pltpu.VMEM((NHALF, CHUNK), jnp.int32),
        pltpu.VMEM((CHUNK, D), jnp.float32),
        pltpu.VMEM((CHUNK, D), jnp.float32),
        pltpu.VMEM_SHARED((N_PAD, D), jnp.float32),
        pltpu.SemaphoreType.DMA,
        pltpu.SemaphoreType.DMA,
    ],
)
def _sc_aggregate(x_hbm, src_hbm, dst_hbm, zeros_hbm, out_hbm,
                  src_v, dst_v, rows0, rows1, agg_sh, gsem0, gsem1):
    cid = lax.axis_index("c")
    sid = lax.axis_index("s")
    wid = sid * NC + cid

    pltpu.sync_copy(zeros_hbm, agg_sh.at[pl.ds(sid * STRIPE, STRIPE)])
    plsc.subcore_barrier()

    for h in range(2):
        pltpu.sync_copy(src_hbm.at[wid, pl.ds(h * NHALF, NHALF)], src_v)
        pltpu.sync_copy(dst_hbm.at[wid, pl.ds(h * NHALF, NHALF)], dst_v)
        pltpu.async_copy(x_hbm.at[src_v.at[0]], rows0, gsem0)

        def body(k, _):
            i0 = 2 * k
            i1 = 2 * k + 1
            i2 = 2 * k + 2
            pltpu.make_async_copy(x_hbm.at[src_v.at[i0]], rows0, gsem0).wait()
            pltpu.async_copy(x_hbm.at[src_v.at[i1]], rows1, gsem1)
            pltpu.sync_copy(rows0, agg_sh.at[dst_v.at[i0]], add=True)
            pltpu.make_async_copy(x_hbm.at[src_v.at[i1]], rows1, gsem1).wait()

            @pl.when(i2 < NHALF)
            def _():
                pltpu.async_copy(x_hbm.at[src_v.at[i2]], rows0, gsem0)

            pltpu.sync_copy(rows1, agg_sh.at[dst_v.at[i1]], add=True)
            return ()

        lax.fori_loop(0, NHALF // 2, body, (), unroll=False)

    plsc.subcore_barrier()
    pltpu.sync_copy(
        agg_sh.at[pl.ds(sid * STRIPE, STRIPE)],
        out_hbm.at[cid, pl.ds(sid * STRIPE, STRIPE)],
    )


_BLK = 632


def _tc_linear_body(agg_ref, w_ref, b_ref, o_ref):
    a = agg_ref[0] + agg_ref[1]
    h = lax.dot_general(a, w_ref[...], (((1,), (1,)), ((), ())),
                        preferred_element_type=jnp.float32)
    o_ref[...] = jnp.maximum(h + b_ref[...], 0.0)


def _tc_linear(agg2, W, b):
    return pl.pallas_call(
        _tc_linear_body,
        grid=(N_PAD // _BLK,),
        in_specs=[
            pl.BlockSpec((NC, _BLK, D), lambda i: (0, i, 0)),
            pl.BlockSpec((D, D), lambda i: (0, 0)),
            pl.BlockSpec((1, D), lambda i: (0, 0)),
        ],
        out_specs=pl.BlockSpec((_BLK, D), lambda i: (i, 0)),
        out_shape=jax.ShapeDtypeStruct((N_PAD, D), jnp.float32),
    )(agg2, W, b.reshape(1, D))


def kernel(x, edge_index, W, b):
    ei = edge_index.astype(jnp.int32)
    pad = E_PAD - E
    pad_dst = DST_PAD + jnp.arange(pad, dtype=jnp.int32) % (N_PAD - N)
    src = jnp.concatenate([ei[0], jnp.zeros((pad,), jnp.int32)])
    dst = jnp.concatenate([ei[1], pad_dst])
    src = src.reshape(NW, NITER, CHUNK)
    dst = dst.reshape(NW, NITER, CHUNK)
    zeros = jnp.zeros((STRIPE, D), jnp.float32)
    agg2 = _sc_aggregate(x, src, dst, zeros)
    return _tc_linear(agg2, W, b)[:N]
